# initial kernel scaffold (unmeasured)
import jax
import jax.numpy as jnp
from jax import lax
from jax.experimental import pallas as pl
from jax.experimental.pallas import tpu as pltpu

N_DEV = 16
SQ = 2048
D_MODEL = 1024
H_LOC = 8
DH = 128
D_HEADS = H_LOC * DH
BLK = 64
NRES = 4
GROUP = SQ // NRES
NBPG = SQ // BLK // NRES
CHUNK = SQ // N_DEV
SCALE = 0.08838834764831843


def _body(x_ref, wq_ref, k_ref, v_ref, wo_ref, out_ref,
          qg_ref, kg_ref, vg_ref, ctxg_ref, stag_ref,
          ss_rs, rs_rs, ss_ag, rs_ag):
    my = lax.axis_index("i")
    left = lax.rem(my + N_DEV - 1, N_DEV)
    right = lax.rem(my + 1, N_DEV)

    qg = jax.lax.dot_general(
        x_ref[:, :], wq_ref[:, :], (((1,), (0,)), ((), ())),
        preferred_element_type=jnp.float32,
    ).astype(jnp.bfloat16)
    for r in range(NRES):
        for b in range(NBPG):
            gs = r * GROUP + b * BLK
            ns = (b * NRES + r) * BLK
            qg_ref[gs:gs + BLK, :] = qg[ns:ns + BLK, :]
            kg_ref[gs:gs + BLK, :] = k_ref[ns:ns + BLK, :]
            vg_ref[gs:gs + BLK, :] = v_ref[ns:ns + BLK, :]

    for r in range(NRES):
        for h in range(H_LOC):
            q = qg_ref[r * GROUP:(r + 1) * GROUP, h * DH:(h + 1) * DH]
            k = kg_ref[r * GROUP:(r + 1) * GROUP, h * DH:(h + 1) * DH]
            v = vg_ref[r * GROUP:(r + 1) * GROUP, h * DH:(h + 1) * DH]
            s = jax.lax.dot_general(
                q, k, (((1,), (1,)), ((), ())),
                preferred_element_type=jnp.float32,
            ) * SCALE
            m = jnp.max(s, axis=1, keepdims=True)
            w = jnp.exp(s - m)
            w = w / jnp.sum(w, axis=1, keepdims=True)
            c = jax.lax.dot_general(
                w.astype(jnp.bfloat16), v, (((1,), (0,)), ((), ())),
                preferred_element_type=jnp.float32,
            )
            ctxg_ref[r * GROUP:(r + 1) * GROUP, h * DH:(h + 1) * DH] = (
                c.astype(jnp.bfloat16))

    part = jax.lax.dot_general(
        ctxg_ref[:, :], wo_ref[:, :], (((1,), (0,)), ((), ())),
        preferred_element_type=jnp.float32,
    )
    for r in range(NRES):
        for b in range(NBPG):
            gs = r * GROUP + b * BLK
            ns = (b * NRES + r) * BLK
            out_ref[ns:ns + BLK, :] = part[gs:gs + BLK, :]

    barrier = pltpu.get_barrier_semaphore()
    for nbr in (left, right):
        pl.semaphore_signal(barrier, inc=1, device_id=(nbr,),
                            device_id_type=pl.DeviceIdType.MESH)
    pl.semaphore_wait(barrier, 2)

    for s in range(N_DEV - 1):
        sc = lax.rem(my - s + N_DEV, N_DEV)
        rc = lax.rem(my - s - 1 + N_DEV, N_DEV)
        rdma = pltpu.make_async_remote_copy(
            src_ref=out_ref.at[pl.ds(sc * CHUNK, CHUNK), :],
            dst_ref=stag_ref.at[s],
            send_sem=ss_rs.at[s],
            recv_sem=rs_rs.at[s],
            device_id=(right,),
            device_id_type=pl.DeviceIdType.MESH,
        )
        rdma.start()
        rdma.wait()
        out_ref[pl.ds(rc * CHUNK, CHUNK), :] = (
            out_ref[pl.ds(rc * CHUNK, CHUNK), :] + stag_ref[s])

    for t in range(N_DEV - 1):
        sc = lax.rem(my + 1 - t + N_DEV, N_DEV)
        rdma = pltpu.make_async_remote_copy(
            src_ref=out_ref.at[pl.ds(sc * CHUNK, CHUNK), :],
            dst_ref=out_ref.at[pl.ds(sc * CHUNK, CHUNK), :],
            send_sem=ss_ag.at[t],
            recv_sem=rs_ag.at[t],
            device_id=(right,),
            device_id_type=pl.DeviceIdType.MESH,
        )
        rdma.start()
        rdma.wait()


def kernel(x, Wq, K_ext, V_ext, Wo):
    i = lax.axis_index("i")
    x2 = x[0].astype(jnp.bfloat16)
    wq_l = lax.dynamic_slice_in_dim(
        Wq, i * D_HEADS, D_HEADS, axis=1).astype(jnp.bfloat16)
    wo_l = lax.dynamic_slice_in_dim(
        Wo, i * D_HEADS, D_HEADS, axis=0).astype(jnp.bfloat16)
    k2 = K_ext[0].reshape(SQ, D_HEADS).astype(jnp.bfloat16)
    v2 = V_ext[0].reshape(SQ, D_HEADS).astype(jnp.bfloat16)

    out = pl.pallas_call(
        _body,
        out_shape=jax.ShapeDtypeStruct((SQ, D_MODEL), jnp.float32),
        in_specs=[pl.BlockSpec(memory_space=pltpu.VMEM)] * 5,
        out_specs=pl.BlockSpec(memory_space=pltpu.VMEM),
        scratch_shapes=[
            pltpu.VMEM((SQ, D_HEADS), jnp.bfloat16),
            pltpu.VMEM((SQ, D_HEADS), jnp.bfloat16),
            pltpu.VMEM((SQ, D_HEADS), jnp.bfloat16),
            pltpu.VMEM((SQ, D_HEADS), jnp.bfloat16),
            pltpu.VMEM((N_DEV - 1, CHUNK, D_MODEL), jnp.float32),
            pltpu.SemaphoreType.DMA((N_DEV - 1,)),
            pltpu.SemaphoreType.DMA((N_DEV - 1,)),
            pltpu.SemaphoreType.DMA((N_DEV - 1,)),
            pltpu.SemaphoreType.DMA((N_DEV - 1,)),
        ],
        compiler_params=pltpu.CompilerParams(collective_id=0),
    )(x2, wq_l, k2, v2, wo_l)
    return out[None]


# baseline (device time: 289023 ns/iter reference)
import jax
import jax.numpy as jnp
from jax import lax
from jax.experimental import pallas as pl
from jax.experimental.pallas import tpu as pltpu

N_DEV = 16
SQ = 2048
D_MODEL = 1024
H_LOC = 8
DH = 128
D_HEADS = H_LOC * DH
BLK = 64
NRES = 4
GROUP = SQ // NRES
NBPG = SQ // BLK // NRES
CHUNK = SQ // N_DEV
SCALE = 0.08838834764831843


def _body(x_ref, wq_ref, k_ref, v_ref, wo_ref, out_ref,
          qg_ref, kg_ref, vg_ref, ctxg_ref, stag_ref,
          ss_rs, rs_rs, ss_ag, rs_ag):
    my = lax.axis_index("i")
    left = lax.rem(my + N_DEV - 1, N_DEV)
    right = lax.rem(my + 1, N_DEV)

    qg = jax.lax.dot_general(
        x_ref[:, :], wq_ref[:, :], (((1,), (0,)), ((), ())),
        preferred_element_type=jnp.float32,
    ).astype(jnp.bfloat16)
    for r in range(NRES):
        for b in range(NBPG):
            gs = r * GROUP + b * BLK
            ns = (b * NRES + r) * BLK
            qg_ref[gs:gs + BLK, :] = qg[ns:ns + BLK, :]
            kg_ref[gs:gs + BLK, :] = k_ref[ns:ns + BLK, :]
            vg_ref[gs:gs + BLK, :] = v_ref[ns:ns + BLK, :]

    for r in range(NRES):
        for h in range(H_LOC):
            q = qg_ref[r * GROUP:(r + 1) * GROUP, h * DH:(h + 1) * DH]
            k = kg_ref[r * GROUP:(r + 1) * GROUP, h * DH:(h + 1) * DH]
            v = vg_ref[r * GROUP:(r + 1) * GROUP, h * DH:(h + 1) * DH]
            s = jax.lax.dot_general(
                q, k, (((1,), (1,)), ((), ())),
                preferred_element_type=jnp.float32,
            ) * SCALE
            m = jnp.max(s, axis=1, keepdims=True)
            w = jnp.exp(s - m)
            w = w / jnp.sum(w, axis=1, keepdims=True)
            c = jax.lax.dot_general(
                w.astype(jnp.bfloat16), v, (((1,), (0,)), ((), ())),
                preferred_element_type=jnp.float32,
            )
            ctxg_ref[r * GROUP:(r + 1) * GROUP, h * DH:(h + 1) * DH] = (
                c.astype(jnp.bfloat16))

    part = jax.lax.dot_general(
        ctxg_ref[:, :], wo_ref[:, :], (((1,), (0,)), ((), ())),
        preferred_element_type=jnp.float32,
    )
    for r in range(NRES):
        for b in range(NBPG):
            gs = r * GROUP + b * BLK
            ns = (b * NRES + r) * BLK
            out_ref[ns:ns + BLK, :] = part[gs:gs + BLK, :]

    barrier = pltpu.get_barrier_semaphore()
    for nbr in (left, right):
        pl.semaphore_signal(barrier, inc=1, device_id=(nbr,),
                            device_id_type=pl.DeviceIdType.MESH)
    pl.semaphore_wait(barrier, 2)

    for s in range(N_DEV - 1):
        sc = lax.rem(my - s + N_DEV, N_DEV)
        rc = lax.rem(my - s - 1 + N_DEV, N_DEV)
        rdma = pltpu.make_async_remote_copy(
            src_ref=out_ref.at[pl.ds(sc * CHUNK, CHUNK), :],
            dst_ref=stag_ref.at[s],
            send_sem=ss_rs.at[s],
            recv_sem=rs_rs.at[s],
            device_id=(right,),
            device_id_type=pl.DeviceIdType.MESH,
        )
        rdma.start()
        rdma.wait()
        out_ref[pl.ds(rc * CHUNK, CHUNK), :] = (
            out_ref[pl.ds(rc * CHUNK, CHUNK), :] + stag_ref[s])

    for t in range(N_DEV - 1):
        sc = lax.rem(my + 1 - t + N_DEV, N_DEV)
        rdma = pltpu.make_async_remote_copy(
            src_ref=out_ref.at[pl.ds(sc * CHUNK, CHUNK), :],
            dst_ref=out_ref.at[pl.ds(sc * CHUNK, CHUNK), :],
            send_sem=ss_ag.at[t],
            recv_sem=rs_ag.at[t],
            device_id=(right,),
            device_id_type=pl.DeviceIdType.MESH,
        )
        rdma.start()
        rdma.wait()


def kernel(x, Wq, K_ext, V_ext, Wo):
    i = lax.axis_index("i")
    x2 = x[0].astype(jnp.bfloat16)
    wq_l = lax.dynamic_slice_in_dim(
        Wq, i * D_HEADS, D_HEADS, axis=1).astype(jnp.bfloat16)
    wo_l = lax.dynamic_slice_in_dim(
        Wo, i * D_HEADS, D_HEADS, axis=0).astype(jnp.bfloat16)
    k2 = K_ext[0].reshape(SQ, D_HEADS).astype(jnp.bfloat16)
    v2 = V_ext[0].reshape(SQ, D_HEADS).astype(jnp.bfloat16)

    out = pl.pallas_call(
        _body,
        out_shape=jax.ShapeDtypeStruct((SQ, D_MODEL), jnp.float32),
        in_specs=[pl.BlockSpec(memory_space=pltpu.VMEM)] * 5,
        out_specs=pl.BlockSpec(memory_space=pltpu.VMEM),
        scratch_shapes=[
            pltpu.VMEM((SQ, D_HEADS), jnp.bfloat16),
            pltpu.VMEM((SQ, D_HEADS), jnp.bfloat16),
            pltpu.VMEM((SQ, D_HEADS), jnp.bfloat16),
            pltpu.VMEM((SQ, D_HEADS), jnp.bfloat16),
            pltpu.VMEM((N_DEV - 1, CHUNK, D_MODEL), jnp.float32),
            pltpu.SemaphoreType.DMA((N_DEV - 1,)),
            pltpu.SemaphoreType.DMA((N_DEV - 1,)),
            pltpu.SemaphoreType.DMA((N_DEV - 1,)),
            pltpu.SemaphoreType.DMA((N_DEV - 1,)),
        ],
        compiler_params=pltpu.CompilerParams(
            collective_id=0, vmem_limit_bytes=100 * 1024 * 1024),
    )(x2, wq_l, k2, v2, wo_l)
    return out[None]


# device time: 165322 ns/iter; 1.7482x vs baseline; 1.7482x over previous
import jax
import jax.numpy as jnp
from jax import lax
from jax.experimental import pallas as pl
from jax.experimental.pallas import tpu as pltpu

N_DEV = 16
SQ = 2048
D_MODEL = 1024
H_LOC = 8
DH = 128
D_HEADS = H_LOC * DH
BLK = 64
NRES = 4
GROUP = SQ // NRES
NBPG = SQ // BLK // NRES
SCALE = 0.08838834764831843

MASKS = (4, 1, 2, 8)
HALVES = (1024, 512, 256, 128)


def _body(x_ref, wq_ref, k_ref, v_ref, wo_ref, out_ref,
          qg_ref, kg_ref, vg_ref, ctxg_ref,
          st0, st1, st2, st3, rs_ss, rs_rs, ag_ss, ag_rs):
    my = lax.axis_index("i")
    stags = (st0, st1, st2, st3)

    qg = jax.lax.dot_general(
        x_ref[:, :], wq_ref[:, :], (((1,), (0,)), ((), ())),
        preferred_element_type=jnp.float32,
    ).astype(jnp.bfloat16)
    for r in range(NRES):
        for b in range(NBPG):
            gs = r * GROUP + b * BLK
            ns = (b * NRES + r) * BLK
            qg_ref[gs:gs + BLK, :] = qg[ns:ns + BLK, :]
            kg_ref[gs:gs + BLK, :] = k_ref[ns:ns + BLK, :]
            vg_ref[gs:gs + BLK, :] = v_ref[ns:ns + BLK, :]

    for r in range(NRES):
        for h in range(H_LOC):
            q = qg_ref[r * GROUP:(r + 1) * GROUP, h * DH:(h + 1) * DH]
            k = kg_ref[r * GROUP:(r + 1) * GROUP, h * DH:(h + 1) * DH]
            v = vg_ref[r * GROUP:(r + 1) * GROUP, h * DH:(h + 1) * DH]
            s = jax.lax.dot_general(
                q, k, (((1,), (1,)), ((), ())),
                preferred_element_type=jnp.float32,
            ) * SCALE
            m = jnp.max(s, axis=1, keepdims=True)
            w = jnp.exp(s - m)
            w = w / jnp.sum(w, axis=1, keepdims=True)
            c = jax.lax.dot_general(
                w.astype(jnp.bfloat16), v, (((1,), (0,)), ((), ())),
                preferred_element_type=jnp.float32,
            )
            ctxg_ref[r * GROUP:(r + 1) * GROUP, h * DH:(h + 1) * DH] = (
                c.astype(jnp.bfloat16))

    part = jax.lax.dot_general(
        ctxg_ref[:, :], wo_ref[:, :], (((1,), (0,)), ((), ())),
        preferred_element_type=jnp.float32,
    ).astype(jnp.bfloat16)
    for r in range(NRES):
        for b in range(NBPG):
            gs = r * GROUP + b * BLK
            ns = (b * NRES + r) * BLK
            out_ref[ns:ns + BLK, :] = part[gs:gs + BLK, :]

    barrier = pltpu.get_barrier_semaphore()
    for mask in MASKS:
        pl.semaphore_signal(
            barrier, inc=1,
            device_id=(jnp.bitwise_xor(my, mask),),
            device_id_type=pl.DeviceIdType.MESH)
    pl.semaphore_wait(barrier, len(MASKS))

    a = jnp.int32(0)
    for kk in range(4):
        mask, h = MASKS[kk], HALVES[kk]
        bit = lax.rem(lax.div(my, mask), 2)
        partner = jnp.bitwise_xor(my, mask)
        send_start = pl.multiple_of(a + jnp.where(bit == 0, h, 0), HALVES[3])
        keep_start = pl.multiple_of(a + jnp.where(bit == 0, 0, h), HALVES[3])
        rdma = pltpu.make_async_remote_copy(
            src_ref=out_ref.at[pl.ds(send_start, h), :],
            dst_ref=stags[kk],
            send_sem=rs_ss.at[kk],
            recv_sem=rs_rs.at[kk],
            device_id=(partner,),
            device_id_type=pl.DeviceIdType.MESH,
        )
        rdma.start()
        rdma.wait()
        out_ref[pl.ds(keep_start, h), :] = (
            out_ref[pl.ds(keep_start, h), :] + stags[kk][:, :])
        a = keep_start

    for kk in (3, 2, 1, 0):
        mask, h = MASKS[kk], HALVES[kk]
        bit = lax.rem(lax.div(my, mask), 2)
        partner = jnp.bitwise_xor(my, mask)
        a = pl.multiple_of(a, HALVES[3])
        rdma = pltpu.make_async_remote_copy(
            src_ref=out_ref.at[pl.ds(a, h), :],
            dst_ref=out_ref.at[pl.ds(a, h), :],
            send_sem=ag_ss.at[kk],
            recv_sem=ag_rs.at[kk],
            device_id=(partner,),
            device_id_type=pl.DeviceIdType.MESH,
        )
        rdma.start()
        rdma.wait()
        a = jnp.where(bit == 0, a, a - h)


def kernel(x, Wq, K_ext, V_ext, Wo):
    i = lax.axis_index("i")
    x2 = x[0].astype(jnp.bfloat16)
    wq_l = lax.dynamic_slice_in_dim(
        Wq, i * D_HEADS, D_HEADS, axis=1).astype(jnp.bfloat16)
    wo_l = lax.dynamic_slice_in_dim(
        Wo, i * D_HEADS, D_HEADS, axis=0).astype(jnp.bfloat16)
    k2 = K_ext[0].reshape(SQ, D_HEADS).astype(jnp.bfloat16)
    v2 = V_ext[0].reshape(SQ, D_HEADS).astype(jnp.bfloat16)

    out = pl.pallas_call(
        _body,
        out_shape=jax.ShapeDtypeStruct((SQ, D_MODEL), jnp.bfloat16),
        in_specs=[pl.BlockSpec(memory_space=pltpu.VMEM)] * 5,
        out_specs=pl.BlockSpec(memory_space=pltpu.VMEM),
        scratch_shapes=[
            pltpu.VMEM((SQ, D_HEADS), jnp.bfloat16),
            pltpu.VMEM((SQ, D_HEADS), jnp.bfloat16),
            pltpu.VMEM((SQ, D_HEADS), jnp.bfloat16),
            pltpu.VMEM((SQ, D_HEADS), jnp.bfloat16),
            pltpu.VMEM((HALVES[0], D_MODEL), jnp.bfloat16),
            pltpu.VMEM((HALVES[1], D_MODEL), jnp.bfloat16),
            pltpu.VMEM((HALVES[2], D_MODEL), jnp.bfloat16),
            pltpu.VMEM((HALVES[3], D_MODEL), jnp.bfloat16),
            pltpu.SemaphoreType.DMA((4,)),
            pltpu.SemaphoreType.DMA((4,)),
            pltpu.SemaphoreType.DMA((4,)),
            pltpu.SemaphoreType.DMA((4,)),
        ],
        compiler_params=pltpu.CompilerParams(
            collective_id=0, vmem_limit_bytes=100 * 1024 * 1024),
    )(x2, wq_l, k2, v2, wo_l)
    return out[None]
